# baseline (speedup 1.0000x reference)
import jax
import jax.numpy as jnp
from jax import lax
from jax.experimental import pallas as pl
from jax.experimental.pallas import tpu as pltpu
from jax.experimental.pallas import tpu_sc as plsc

BATCH = 4096
HIST = 200
EMBED = 32

NC = 2
NS = 16
NW = NC * NS

B = BATCH * HIST
CH = 128
ROWS_PER_W = B // NW
NCH = ROWS_PER_W // CH
K = 10
NT = NCH // K
BLOCK_ROWS = K * CH


def _gather_body(table_hbm, idx_hbm, out_hbm, idx_v, rows0, rows1, g0, g1,
                 s0, s1):
    wid = lax.axis_index("s") * NC + lax.axis_index("c")
    chunk_base = wid * NCH
    row_base = wid * ROWS_PER_W
    pltpu.sync_copy(idx_hbm.at[pl.ds(chunk_base, NCH)], idx_v)

    def fire_block(t, rows_v, g_sem):
        descs = []
        for j in range(K):
            descs.append(pltpu.async_copy(
                table_hbm.at[idx_v.at[t * K + j]],
                rows_v.at[pl.ds(j * CH, CH)], g_sem))
        for d in descs:
            d.wait()

    def store_block(t, rows_v, s_sem):
        pltpu.async_copy(
            rows_v, out_hbm.at[pl.ds(row_base + t * BLOCK_ROWS, BLOCK_ROWS)],
            s_sem)

    def wait_store(rows_v, s_sem):
        pltpu.make_async_copy(
            rows_v, out_hbm.at[pl.ds(row_base, BLOCK_ROWS)], s_sem).wait()

    @pl.loop(0, NT, step=2)
    def _(tp):
        @pl.when(tp >= 2)
        def _():
            wait_store(rows0, s0)
        fire_block(tp, rows0, g0)
        store_block(tp, rows0, s0)

        @pl.when(tp >= 2)
        def _():
            wait_store(rows1, s1)
        fire_block(tp + 1, rows1, g1)
        store_block(tp + 1, rows1, s1)

    wait_store(rows0, s0)
    wait_store(rows1, s1)


def _format_body(bo_hbm, out_hbm, idx0, idx1, binb0, binb1, outt0, outt1,
                 g0, g1, s0, s1):
    w = lax.axis_index("s") * NC + lax.axis_index("c")
    iota = lax.iota(jnp.int32, 16)

    def fire(u, idx_v, binb, g_sem):
        base = 6400 * w + u
        for kc in range(8):
            idx_v[pl.ds(16 * kc, 16)] = base + 50 * (16 * kc + iota)
        pltpu.async_copy(bo_hbm.at[idx_v], binb, g_sem)

    def wait_gather(idx_v, binb, g_sem):
        pltpu.make_async_copy(bo_hbm.at[idx_v], binb, g_sem).wait()

    def transpose_store(u, binb, outt, s_sem):
        @plsc.parallel_loop(0, 4 * EMBED, unroll=16)
        def _(c):
            col = lax.broadcast(c, (16,))
            for k0 in range(8):
                v = plsc.load_gather(binb, [16 * k0 + iota, col])
                outt[c, pl.ds(16 * k0, 16)] = v
        for hp in range(4):
            pltpu.async_copy(
                outt.at[pl.ds(32 * hp, 32)],
                out_hbm.at[4 * u + hp, slice(None), pl.ds(128 * w, 128)],
                s_sem)

    def wait_stores(outt, s_sem):
        for hp in range(4):
            pltpu.make_async_copy(
                outt.at[pl.ds(32 * hp, 32)],
                out_hbm.at[hp, slice(None), pl.ds(128 * w, 128)], s_sem).wait()

    fire(0, idx0, binb0, g0)

    @pl.loop(0, HIST // 4, step=2)
    def _(up):
        wait_gather(idx0, binb0, g0)
        fire(up + 1, idx1, binb1, g1)

        @pl.when(up >= 2)
        def _():
            wait_stores(outt0, s0)
        transpose_store(up, binb0, outt0, s0)

        wait_gather(idx1, binb1, g1)

        @pl.when(up + 2 < HIST // 4)
        def _():
            fire(up + 2, idx0, binb0, g0)

        @pl.when(up >= 2)
        def _():
            wait_stores(outt1, s1)
        transpose_store(up + 1, binb1, outt1, s1)

    wait_stores(outt0, s0)
    wait_stores(outt1, s1)


@jax.jit
def _run(table, idx2d):
    mesh = plsc.VectorSubcoreMesh(
        core_axis_name="c", subcore_axis_name="s",
        num_cores=NC, num_subcores=NS,
    )
    gather = pl.kernel(
        _gather_body,
        out_type=jax.ShapeDtypeStruct((B, EMBED), jnp.float32),
        mesh=mesh,
        scratch_types=[
            pltpu.VMEM((NCH, CH), jnp.int32),
            pltpu.VMEM((BLOCK_ROWS, EMBED), jnp.float32),
            pltpu.VMEM((BLOCK_ROWS, EMBED), jnp.float32),
            pltpu.SemaphoreType.DMA,
            pltpu.SemaphoreType.DMA,
            pltpu.SemaphoreType.DMA,
            pltpu.SemaphoreType.DMA,
        ],
        compiler_params=pltpu.CompilerParams(use_tc_tiling_on_sc=False),
    )
    fmt = pl.kernel(
        _format_body,
        out_type=jax.ShapeDtypeStruct((HIST, EMBED, BATCH), jnp.float32),
        mesh=mesh,
        scratch_types=[
            pltpu.VMEM((128,), jnp.int32),
            pltpu.VMEM((128,), jnp.int32),
            pltpu.VMEM((128, 128), jnp.float32),
            pltpu.VMEM((128, 128), jnp.float32),
            pltpu.VMEM((128, 128), jnp.float32),
            pltpu.VMEM((128, 128), jnp.float32),
            pltpu.SemaphoreType.DMA,
            pltpu.SemaphoreType.DMA,
            pltpu.SemaphoreType.DMA,
            pltpu.SemaphoreType.DMA,
        ],
        compiler_params=pltpu.CompilerParams(
            use_tc_tiling_on_sc=True, needs_layout_passes=False),
    )
    bout = gather(table, idx2d)
    outp = fmt(bout.reshape(B // 4, 128))
    return outp.transpose(2, 0, 1)


def kernel(inputs, table):
    idx2d = inputs.reshape(B // CH, CH)
    return _run(table, idx2d)


# speedup vs baseline: 1.1252x; 1.1252x over previous
import jax
import jax.numpy as jnp
from jax import lax
from jax.experimental import pallas as pl
from jax.experimental.pallas import tpu as pltpu
from jax.experimental.pallas import tpu_sc as plsc

BATCH = 4096
HIST = 200
EMBED = 32

NC = 2
NS = 16
NW = NC * NS

B = BATCH * HIST
CH = 128
ROWS_PER_W = B // NW
NCH = ROWS_PER_W // CH
K = 10
NT = NCH // K
BLOCK_ROWS = K * CH


def _gather_body(table_hbm, idx_hbm, out_hbm, idx_v, rows0, rows1, g0, g1,
                 s0, s1):
    wid = lax.axis_index("s") * NC + lax.axis_index("c")
    chunk_base = wid * NCH
    row_base = wid * ROWS_PER_W
    pltpu.sync_copy(idx_hbm.at[pl.ds(chunk_base, NCH)], idx_v)

    def fire_block(t, rows_v, g_sem):
        descs = []
        for j in range(K):
            descs.append(pltpu.async_copy(
                table_hbm.at[idx_v.at[t * K + j]],
                rows_v.at[pl.ds(j * CH, CH)], g_sem))
        for d in descs:
            d.wait()

    def store_block(t, rows_v, s_sem):
        pltpu.async_copy(
            rows_v, out_hbm.at[pl.ds(row_base + t * BLOCK_ROWS, BLOCK_ROWS)],
            s_sem)

    def wait_store(rows_v, s_sem):
        pltpu.make_async_copy(
            rows_v, out_hbm.at[pl.ds(row_base, BLOCK_ROWS)], s_sem).wait()

    @pl.loop(0, NT, step=2)
    def _(tp):
        @pl.when(tp >= 2)
        def _():
            wait_store(rows0, s0)
        fire_block(tp, rows0, g0)
        store_block(tp, rows0, s0)

        @pl.when(tp >= 2)
        def _():
            wait_store(rows1, s1)
        fire_block(tp + 1, rows1, g1)
        store_block(tp + 1, rows1, s1)

    wait_store(rows0, s0)
    wait_store(rows1, s1)


def _format_body(bo_hbm, out_hbm, idx0, idx1, binb0, binb1, outt0, outt1,
                 g0, g1, s0, s1):
    w = lax.axis_index("s") * NC + lax.axis_index("c")
    iota = lax.iota(jnp.int32, 16)

    def fire(u, idx_v, binb, g_sem):
        base = 6400 * w + u
        for kc in range(8):
            idx_v[pl.ds(16 * kc, 16)] = base + 50 * (16 * kc + iota)
        pltpu.async_copy(bo_hbm.at[idx_v], binb, g_sem)

    def wait_gather(idx_v, binb, g_sem):
        pltpu.make_async_copy(bo_hbm.at[idx_v], binb, g_sem).wait()

    k_pat = lax.shift_right_logical(iota, 3)
    c_pat = lax.bitwise_and(iota, 7)

    def transpose_store(u, binb, outt, s_sem):
        @plsc.parallel_loop(0, 4 * EMBED // 8, unroll=4)
        def _(cb):
            c0 = 8 * cb
            cols = c0 + c_pat
            for k0 in range(0, 128, 2):
                rows = k0 + k_pat
                v = plsc.load_gather(binb, [rows, cols])
                plsc.store_scatter(outt, [cols, rows], v)
        for hp in range(4):
            pltpu.async_copy(
                outt.at[pl.ds(32 * hp, 32)],
                out_hbm.at[4 * u + hp, slice(None), pl.ds(128 * w, 128)],
                s_sem)

    def wait_stores(outt, s_sem):
        for hp in range(4):
            pltpu.make_async_copy(
                outt.at[pl.ds(32 * hp, 32)],
                out_hbm.at[hp, slice(None), pl.ds(128 * w, 128)], s_sem).wait()

    fire(0, idx0, binb0, g0)

    @pl.loop(0, HIST // 4, step=2)
    def _(up):
        wait_gather(idx0, binb0, g0)
        fire(up + 1, idx1, binb1, g1)

        @pl.when(up >= 2)
        def _():
            wait_stores(outt0, s0)
        transpose_store(up, binb0, outt0, s0)

        wait_gather(idx1, binb1, g1)

        @pl.when(up + 2 < HIST // 4)
        def _():
            fire(up + 2, idx0, binb0, g0)

        @pl.when(up >= 2)
        def _():
            wait_stores(outt1, s1)
        transpose_store(up + 1, binb1, outt1, s1)

    wait_stores(outt0, s0)
    wait_stores(outt1, s1)


@jax.jit
def _run(table, idx2d):
    mesh = plsc.VectorSubcoreMesh(
        core_axis_name="c", subcore_axis_name="s",
        num_cores=NC, num_subcores=NS,
    )
    gather = pl.kernel(
        _gather_body,
        out_type=jax.ShapeDtypeStruct((B, EMBED), jnp.float32),
        mesh=mesh,
        scratch_types=[
            pltpu.VMEM((NCH, CH), jnp.int32),
            pltpu.VMEM((BLOCK_ROWS, EMBED), jnp.float32),
            pltpu.VMEM((BLOCK_ROWS, EMBED), jnp.float32),
            pltpu.SemaphoreType.DMA,
            pltpu.SemaphoreType.DMA,
            pltpu.SemaphoreType.DMA,
            pltpu.SemaphoreType.DMA,
        ],
        compiler_params=pltpu.CompilerParams(use_tc_tiling_on_sc=False),
    )
    fmt = pl.kernel(
        _format_body,
        out_type=jax.ShapeDtypeStruct((HIST, EMBED, BATCH), jnp.float32),
        mesh=mesh,
        scratch_types=[
            pltpu.VMEM((128,), jnp.int32),
            pltpu.VMEM((128,), jnp.int32),
            pltpu.VMEM((128, 128), jnp.float32),
            pltpu.VMEM((128, 128), jnp.float32),
            pltpu.VMEM((128, 128), jnp.float32),
            pltpu.VMEM((128, 128), jnp.float32),
            pltpu.SemaphoreType.DMA,
            pltpu.SemaphoreType.DMA,
            pltpu.SemaphoreType.DMA,
            pltpu.SemaphoreType.DMA,
        ],
        compiler_params=pltpu.CompilerParams(
            use_tc_tiling_on_sc=True, needs_layout_passes=False),
    )
    bout = gather(table, idx2d)
    outp = fmt(bout.reshape(B // 4, 128))
    return outp.transpose(2, 0, 1)


def kernel(inputs, table):
    idx2d = inputs.reshape(B // CH, CH)
    return _run(table, idx2d)


# speedup vs baseline: 1.2716x; 1.1301x over previous
import jax
import jax.numpy as jnp
from jax import lax
from jax.experimental import pallas as pl
from jax.experimental.pallas import tpu as pltpu
from jax.experimental.pallas import tpu_sc as plsc

BATCH = 4096
HIST = 200
EMBED = 32

NC = 2
NS = 16
NW = NC * NS

B = BATCH * HIST
CH = 128
ROWS_PER_W = B // NW
NCH = ROWS_PER_W // CH
K = 10
NT = NCH // K
BLOCK_ROWS = K * CH


def _gather_body(table_hbm, idx_hbm, out_hbm, idx_v, rows0, rows1, g0, g1,
                 s0, s1):
    wid = lax.axis_index("s") * NC + lax.axis_index("c")
    chunk_base = wid * NCH
    row_base = wid * ROWS_PER_W
    pltpu.sync_copy(idx_hbm.at[pl.ds(chunk_base, NCH)], idx_v)

    def fire_block(t, rows_v, g_sem):
        descs = []
        for j in range(K):
            descs.append(pltpu.async_copy(
                table_hbm.at[idx_v.at[t * K + j]],
                rows_v.at[pl.ds(j * CH, CH)], g_sem))
        for d in descs:
            d.wait()

    def store_block(t, rows_v, s_sem):
        pltpu.async_copy(
            rows_v, out_hbm.at[pl.ds(row_base + t * BLOCK_ROWS, BLOCK_ROWS)],
            s_sem)

    def wait_store(rows_v, s_sem):
        pltpu.make_async_copy(
            rows_v, out_hbm.at[pl.ds(row_base, BLOCK_ROWS)], s_sem).wait()

    @pl.loop(0, NT, step=2)
    def _(tp):
        @pl.when(tp >= 2)
        def _():
            wait_store(rows0, s0)
        fire_block(tp, rows0, g0)
        store_block(tp, rows0, s0)

        @pl.when(tp >= 2)
        def _():
            wait_store(rows1, s1)
        fire_block(tp + 1, rows1, g1)
        store_block(tp + 1, rows1, s1)

    wait_store(rows0, s0)
    wait_store(rows1, s1)


def _format_body(bo_hbm, out_hbm, idx0, idx1, binb0, binb1, outt0, outt1,
                 g0, g1, s0, s1):
    w = lax.axis_index("s") * NC + lax.axis_index("c")
    iota = lax.iota(jnp.int32, 16)

    def fire(u, idx_v, binb, g_sem):
        base = 6400 * w + u
        for kc in range(8):
            idx_v[pl.ds(16 * kc, 16)] = base + 50 * (16 * kc + iota)
        pltpu.async_copy(bo_hbm.at[idx_v], binb, g_sem)

    def wait_gather(idx_v, binb, g_sem):
        pltpu.make_async_copy(bo_hbm.at[idx_v], binb, g_sem).wait()

    k_pat = lax.shift_right_logical(iota, 2)
    c_pat = lax.bitwise_and(iota, 3)

    def transpose_store(u, binb, outt, s_sem):
        @plsc.parallel_loop(0, 4 * EMBED // 4, unroll=8)
        def _(cb):
            c0 = 4 * cb
            cols = c0 + c_pat
            for k0 in range(0, 128, 4):
                rows = k0 + k_pat
                v = plsc.load_gather(binb, [rows, cols])
                plsc.store_scatter(outt, [cols, rows], v)
        for hp in range(4):
            pltpu.async_copy(
                outt.at[pl.ds(32 * hp, 32)],
                out_hbm.at[4 * u + hp, slice(None), pl.ds(128 * w, 128)],
                s_sem)

    def wait_stores(outt, s_sem):
        for hp in range(4):
            pltpu.make_async_copy(
                outt.at[pl.ds(32 * hp, 32)],
                out_hbm.at[hp, slice(None), pl.ds(128 * w, 128)], s_sem).wait()

    fire(0, idx0, binb0, g0)

    @pl.loop(0, HIST // 4, step=2)
    def _(up):
        wait_gather(idx0, binb0, g0)
        fire(up + 1, idx1, binb1, g1)

        @pl.when(up >= 2)
        def _():
            wait_stores(outt0, s0)
        transpose_store(up, binb0, outt0, s0)

        wait_gather(idx1, binb1, g1)

        @pl.when(up + 2 < HIST // 4)
        def _():
            fire(up + 2, idx0, binb0, g0)

        @pl.when(up >= 2)
        def _():
            wait_stores(outt1, s1)
        transpose_store(up + 1, binb1, outt1, s1)

    wait_stores(outt0, s0)
    wait_stores(outt1, s1)


@jax.jit
def _run(table, idx2d):
    mesh = plsc.VectorSubcoreMesh(
        core_axis_name="c", subcore_axis_name="s",
        num_cores=NC, num_subcores=NS,
    )
    gather = pl.kernel(
        _gather_body,
        out_type=jax.ShapeDtypeStruct((B, EMBED), jnp.float32),
        mesh=mesh,
        scratch_types=[
            pltpu.VMEM((NCH, CH), jnp.int32),
            pltpu.VMEM((BLOCK_ROWS, EMBED), jnp.float32),
            pltpu.VMEM((BLOCK_ROWS, EMBED), jnp.float32),
            pltpu.SemaphoreType.DMA,
            pltpu.SemaphoreType.DMA,
            pltpu.SemaphoreType.DMA,
            pltpu.SemaphoreType.DMA,
        ],
        compiler_params=pltpu.CompilerParams(use_tc_tiling_on_sc=False),
    )
    fmt = pl.kernel(
        _format_body,
        out_type=jax.ShapeDtypeStruct((HIST, EMBED, BATCH), jnp.float32),
        mesh=mesh,
        scratch_types=[
            pltpu.VMEM((128,), jnp.int32),
            pltpu.VMEM((128,), jnp.int32),
            pltpu.VMEM((128, 128), jnp.float32),
            pltpu.VMEM((128, 128), jnp.float32),
            pltpu.VMEM((128, 128), jnp.float32),
            pltpu.VMEM((128, 128), jnp.float32),
            pltpu.SemaphoreType.DMA,
            pltpu.SemaphoreType.DMA,
            pltpu.SemaphoreType.DMA,
            pltpu.SemaphoreType.DMA,
        ],
        compiler_params=pltpu.CompilerParams(
            use_tc_tiling_on_sc=True, needs_layout_passes=False),
    )
    bout = gather(table, idx2d)
    outp = fmt(bout.reshape(B // 4, 128))
    return outp.transpose(2, 0, 1)


def kernel(inputs, table):
    idx2d = inputs.reshape(B // CH, CH)
    return _run(table, idx2d)


# speedup vs baseline: 1.9603x; 1.5416x over previous
import jax
import jax.numpy as jnp
from jax import lax
from jax.experimental import pallas as pl
from jax.experimental.pallas import tpu as pltpu
from jax.experimental.pallas import tpu_sc as plsc

BATCH = 4096
HIST = 200
EMBED = 32

NC = 2
NS = 16
NW = NC * NS

B = BATCH * HIST
CH = 128
ROWS_PER_W = B // NW
NCH = ROWS_PER_W // CH
K = 10
NT = NCH // K
BLOCK_ROWS = K * CH


def _gather_body(table_hbm, idx_hbm, out_hbm, idx_v, rows0, rows1, g0, g1,
                 s0, s1):
    wid = lax.axis_index("s") * NC + lax.axis_index("c")
    chunk_base = wid * NCH
    row_base = wid * ROWS_PER_W
    pltpu.sync_copy(idx_hbm.at[pl.ds(chunk_base, NCH)], idx_v)

    def fire_block(t, rows_v, g_sem):
        descs = []
        for j in range(K):
            descs.append(pltpu.async_copy(
                table_hbm.at[idx_v.at[t * K + j]],
                rows_v.at[pl.ds(j * CH, CH)], g_sem))
        for d in descs:
            d.wait()

    def store_block(t, rows_v, s_sem):
        pltpu.async_copy(
            rows_v, out_hbm.at[pl.ds(row_base + t * BLOCK_ROWS, BLOCK_ROWS)],
            s_sem)

    def wait_store(rows_v, s_sem):
        pltpu.make_async_copy(
            rows_v, out_hbm.at[pl.ds(row_base, BLOCK_ROWS)], s_sem).wait()

    @pl.loop(0, NT, step=2)
    def _(tp):
        @pl.when(tp >= 2)
        def _():
            wait_store(rows0, s0)
        fire_block(tp, rows0, g0)
        store_block(tp, rows0, s0)

        @pl.when(tp >= 2)
        def _():
            wait_store(rows1, s1)
        fire_block(tp + 1, rows1, g1)
        store_block(tp + 1, rows1, s1)

    wait_store(rows0, s0)
    wait_store(rows1, s1)


def _table_body(tt_hbm, tail_hbm, at_hbm, inb0, inb1, outb0, outb1,
                g0, g1, s0, s1):
    w = lax.axis_index("s") * NC + lax.axis_index("c")
    iota = lax.iota(jnp.int32, 16)
    k_pat = lax.shift_right_logical(iota, 2)
    c_pat = lax.bitwise_and(iota, 3)

    def fire(j, inb, g_sem):
        pltpu.async_copy(tt_hbm.at[slice(None), pl.ds(128 * j, 128)], inb,
                         g_sem)

    def wait_read(inb, g_sem):
        pltpu.make_async_copy(
            tt_hbm.at[slice(None), pl.ds(0, 128)], inb, g_sem).wait()

    def transpose_store(j, inb, outb, s_sem):
        @plsc.parallel_loop(0, 32, unroll=8)
        def _(x0b):
            x0 = 4 * x0b
            xs = x0 + c_pat
            e0 = lax.rem(x0, 32)
            m0 = lax.div(x0, 32)
            for p0 in range(0, 32, 4):
                ps = p0 + k_pat
                v = plsc.load_gather(inb, [e0 + c_pat, 4 * ps + m0])
                plsc.store_scatter(outb, [ps, xs], v)
        pltpu.async_copy(outb, at_hbm.at[pl.ds(32 * j, 32)], s_sem)

    def wait_store(outb, s_sem):
        pltpu.make_async_copy(outb, at_hbm.at[pl.ds(0, 32)], s_sem).wait()

    fire(w, inb0, g0)

    @pl.loop(0, 244, step=2)
    def _(t):
        wait_read(inb0, g0)
        fire(w + 32 * (t + 1), inb1, g1)

        @pl.when(t >= 2)
        def _():
            wait_store(outb0, s0)
        transpose_store(w + 32 * t, inb0, outb0, s0)

        wait_read(inb1, g1)

        @pl.when(t + 2 < 244)
        def _():
            fire(w + 32 * (t + 2), inb0, g0)

        @pl.when(jnp.logical_and(t + 2 == 244, w < 4))
        def _():
            fire(7808 + w, inb0, g0)

        @pl.when(t >= 2)
        def _():
            wait_store(outb1, s1)
        transpose_store(w + 32 * (t + 1), inb1, outb1, s1)

    @pl.when(w < 4)
    def _():
        wait_read(inb0, g0)
        wait_store(outb0, s0)
        transpose_store(7808 + w, inb0, outb0, s0)
        wait_store(outb0, s0)

    @pl.when(w >= 4)
    def _():
        wait_store(outb0, s0)
    wait_store(outb1, s1)

    @pl.when(w == 31)
    def _():
        pltpu.sync_copy(tail_hbm, at_hbm.at[pl.ds(249984, 16)])


def _format_body(bo_hbm, out_hbm, idx0, idx1, binb0, binb1, outt0, outt1,
                 g0, g1, s0, s1):
    w = lax.axis_index("s") * NC + lax.axis_index("c")
    iota = lax.iota(jnp.int32, 16)

    def fire(u, idx_v, binb, g_sem):
        base = 6400 * w + u
        for kc in range(8):
            idx_v[pl.ds(16 * kc, 16)] = base + 50 * (16 * kc + iota)
        pltpu.async_copy(bo_hbm.at[idx_v], binb, g_sem)

    def wait_gather(idx_v, binb, g_sem):
        pltpu.make_async_copy(bo_hbm.at[idx_v], binb, g_sem).wait()

    k_pat = lax.shift_right_logical(iota, 2)
    c_pat = lax.bitwise_and(iota, 3)

    def transpose_store(u, binb, outt, s_sem):
        @plsc.parallel_loop(0, 4 * EMBED // 4, unroll=8)
        def _(cb):
            c0 = 4 * cb
            cols = c0 + c_pat
            for k0 in range(0, 128, 4):
                rows = k0 + k_pat
                v = plsc.load_gather(binb, [rows, cols])
                plsc.store_scatter(outt, [cols, rows], v)
        for hp in range(4):
            pltpu.async_copy(
                outt.at[pl.ds(32 * hp, 32)],
                out_hbm.at[4 * u + hp, slice(None), pl.ds(128 * w, 128)],
                s_sem)

    def wait_stores(outt, s_sem):
        for hp in range(4):
            pltpu.make_async_copy(
                outt.at[pl.ds(32 * hp, 32)],
                out_hbm.at[hp, slice(None), pl.ds(128 * w, 128)], s_sem).wait()

    fire(0, idx0, binb0, g0)

    @pl.loop(0, HIST // 4, step=2)
    def _(up):
        wait_gather(idx0, binb0, g0)
        fire(up + 1, idx1, binb1, g1)

        @pl.when(up >= 2)
        def _():
            wait_stores(outt0, s0)
        transpose_store(up, binb0, outt0, s0)

        wait_gather(idx1, binb1, g1)

        @pl.when(up + 2 < HIST // 4)
        def _():
            fire(up + 2, idx0, binb0, g0)

        @pl.when(up >= 2)
        def _():
            wait_stores(outt1, s1)
        transpose_store(up + 1, binb1, outt1, s1)

    wait_stores(outt0, s0)
    wait_stores(outt1, s1)


@jax.jit
def _run(table, idx2d):
    mesh = plsc.VectorSubcoreMesh(
        core_axis_name="c", subcore_axis_name="s",
        num_cores=NC, num_subcores=NS,
    )
    gather = pl.kernel(
        _gather_body,
        out_type=jax.ShapeDtypeStruct((B, EMBED), jnp.float32),
        mesh=mesh,
        scratch_types=[
            pltpu.VMEM((NCH, CH), jnp.int32),
            pltpu.VMEM((BLOCK_ROWS, EMBED), jnp.float32),
            pltpu.VMEM((BLOCK_ROWS, EMBED), jnp.float32),
            pltpu.SemaphoreType.DMA,
            pltpu.SemaphoreType.DMA,
            pltpu.SemaphoreType.DMA,
            pltpu.SemaphoreType.DMA,
        ],
        compiler_params=pltpu.CompilerParams(use_tc_tiling_on_sc=False),
    )
    tbl = pl.kernel(
        _table_body,
        out_type=jax.ShapeDtypeStruct((250000, 128), jnp.float32),
        mesh=mesh,
        scratch_types=[
            pltpu.VMEM((32, 128), jnp.float32),
            pltpu.VMEM((32, 128), jnp.float32),
            pltpu.VMEM((32, 128), jnp.float32),
            pltpu.VMEM((32, 128), jnp.float32),
            pltpu.SemaphoreType.DMA,
            pltpu.SemaphoreType.DMA,
            pltpu.SemaphoreType.DMA,
            pltpu.SemaphoreType.DMA,
        ],
        compiler_params=pltpu.CompilerParams(
            use_tc_tiling_on_sc=True, needs_layout_passes=False),
    )
    fmt = pl.kernel(
        _format_body,
        out_type=jax.ShapeDtypeStruct((HIST, EMBED, BATCH), jnp.float32),
        mesh=mesh,
        scratch_types=[
            pltpu.VMEM((128,), jnp.int32),
            pltpu.VMEM((128,), jnp.int32),
            pltpu.VMEM((128, 128), jnp.float32),
            pltpu.VMEM((128, 128), jnp.float32),
            pltpu.VMEM((128, 128), jnp.float32),
            pltpu.VMEM((128, 128), jnp.float32),
            pltpu.SemaphoreType.DMA,
            pltpu.SemaphoreType.DMA,
            pltpu.SemaphoreType.DMA,
            pltpu.SemaphoreType.DMA,
        ],
        compiler_params=pltpu.CompilerParams(
            use_tc_tiling_on_sc=True, needs_layout_passes=False),
    )
    tt = jnp.swapaxes(table, 0, 1)
    tail = table[999936:].reshape(16, 128)
    at = tbl(tt, tail)
    bout = gather(at.reshape(1000000, 32), idx2d)
    outp = fmt(bout.reshape(B // 4, 128))
    return outp.transpose(2, 0, 1)


def kernel(inputs, table):
    idx2d = inputs.reshape(B // CH, CH)
    return _run(table, idx2d)
